# speedup vs baseline: 3.3365x; 3.3365x over previous
import functools

import jax
import jax.numpy as jnp
from jax import lax
from jax.experimental import pallas as pl
from jax.experimental.pallas import tpu as pltpu
from jax.experimental.pallas import tpu_sc as plsc

D = 128
_NC, _NS = 2, 16
_NW = _NC * _NS


@functools.cache
def _make_gather(B: int, V: int):
    b_per_w = B // _NW
    K = 4
    C = K * 128
    n_chunks = b_per_w // C
    idx_rows_per_w = b_per_w // 128

    mesh = plsc.VectorSubcoreMesh(core_axis_name="c", subcore_axis_name="s")

    @functools.partial(
        pl.kernel,
        out_type=jax.ShapeDtypeStruct((B, D), jnp.float32),
        mesh=mesh,
        scratch_types=[
            pltpu.VMEM((K, 128), jnp.int32),
            pltpu.VMEM((C, D), jnp.float32),
            pltpu.SemaphoreType.DMA,
        ],
    )
    def gather_k(idx_hbm, w_hbm, out_hbm, idx_v, rows_v, sem):
        wid = lax.axis_index("s") * _NC + lax.axis_index("c")
        row_base = wid * idx_rows_per_w
        out_base = wid * b_per_w

        def step(i, carry):
            pltpu.sync_copy(idx_hbm.at[pl.ds(row_base + i * K, K)], idx_v)
            cps = [
                pltpu.async_copy(
                    w_hbm.at[idx_v.at[j]],
                    rows_v.at[pl.ds(j * 128, 128)],
                    sem,
                )
                for j in range(K)
            ]
            for cp in cps:
                cp.wait()
            pltpu.sync_copy(rows_v, out_hbm.at[pl.ds(out_base + i * C, C)])
            return carry

        lax.fori_loop(0, n_chunks, step, 0)

    return gather_k


def kernel(input_, weight):
    B = input_.size
    idx = input_.reshape(B // 128, 128).astype(jnp.int32)
    out = _make_gather(B, weight.shape[0])(idx, weight)
    return out.reshape(*input_.shape, D)


# speedup vs baseline: 3.4516x; 1.0345x over previous
import functools

import jax
import jax.numpy as jnp
from jax import lax
from jax.experimental import pallas as pl
from jax.experimental.pallas import tpu as pltpu
from jax.experimental.pallas import tpu_sc as plsc

D = 128
_NC, _NS = 2, 16
_NW = _NC * _NS


@functools.cache
def _make_gather(B: int, V: int):
    b_per_w = B // _NW
    K = 2
    C = K * 128
    n_chunks = b_per_w // C
    idx_rows_per_w = b_per_w // 128

    mesh = plsc.VectorSubcoreMesh(core_axis_name="c", subcore_axis_name="s")

    @functools.partial(
        pl.kernel,
        out_type=jax.ShapeDtypeStruct((B, D), jnp.float32),
        mesh=mesh,
        scratch_types=[
            pltpu.VMEM((idx_rows_per_w, 128), jnp.int32),
            pltpu.VMEM((C, D), jnp.float32),
            pltpu.VMEM((C, D), jnp.float32),
            pltpu.SemaphoreType.DMA,
            pltpu.SemaphoreType.DMA,
            pltpu.SemaphoreType.DMA,
            pltpu.SemaphoreType.DMA,
        ],
    )
    def gather_k(idx_hbm, w_hbm, out_hbm, idx_v, rows0, rows1,
                 sg0, sg1, ss0, ss1):
        wid = lax.axis_index("s") * _NC + lax.axis_index("c")
        out_base = wid * b_per_w

        pltpu.sync_copy(idx_hbm.at[pl.ds(wid * idx_rows_per_w, idx_rows_per_w)],
                        idx_v)

        rows = (rows0, rows1)
        sg = (sg0, sg1)
        ss = (ss0, ss1)

        def fire_gather(chunk, b):
            for j in range(K):
                pltpu.async_copy(
                    w_hbm.at[idx_v.at[chunk * K + j]],
                    rows[b].at[pl.ds(j * 128, 128)],
                    sg[b],
                )

        def wait_gather(b):
            pltpu.make_async_copy(w_hbm.at[pl.ds(0, C)], rows[b], sg[b]).wait()

        def fire_store(chunk, b):
            pltpu.async_copy(rows[b], out_hbm.at[pl.ds(out_base + chunk * C, C)],
                             ss[b])

        def wait_store(b):
            pltpu.make_async_copy(rows[b], out_hbm.at[pl.ds(out_base, C)],
                                  ss[b]).wait()

        fire_gather(0, 0)
        wait_gather(0)
        fire_store(0, 0)
        fire_gather(1, 1)

        def body(p, carry):
            i = 2 + 2 * p
            wait_store(0)
            fire_gather(i, 0)
            wait_gather(1)
            fire_store(i - 1, 1)
            wait_store(1)
            fire_gather(i + 1, 1)
            wait_gather(0)
            fire_store(i, 0)
            return carry

        lax.fori_loop(0, (n_chunks - 2) // 2, body, 0)

        wait_gather(1)
        fire_store(n_chunks - 1, 1)
        wait_store(0)
        wait_store(1)

    return gather_k


def kernel(input_, weight):
    B = input_.size
    idx = input_.reshape(B // 128, 128).astype(jnp.int32)
    out = _make_gather(B, weight.shape[0])(idx, weight)
    return out.reshape(*input_.shape, D)


# speedup vs baseline: 6.3889x; 1.8510x over previous
import functools

import jax
import jax.numpy as jnp
from jax import lax
from jax.experimental import pallas as pl
from jax.experimental.pallas import tpu as pltpu
from jax.experimental.pallas import tpu_sc as plsc

D = 128
_NC, _NS = 2, 16
_NW = _NC * _NS


@functools.cache
def _make_gather(BATCH: int, H: int, V: int):
    b_per_w = BATCH // _NW
    NB = 4
    n_chunks = b_per_w // NB

    mesh = plsc.VectorSubcoreMesh(core_axis_name="c", subcore_axis_name="s")

    @functools.partial(
        pl.kernel,
        out_type=jax.ShapeDtypeStruct((BATCH, H, D), jnp.float32),
        mesh=mesh,
        scratch_types=[
            pltpu.VMEM((b_per_w, H), jnp.int32),
            pltpu.VMEM((NB, H, D), jnp.float32),
            pltpu.VMEM((NB, H, D), jnp.float32),
            pltpu.SemaphoreType.DMA,
            pltpu.SemaphoreType.DMA,
            pltpu.SemaphoreType.DMA,
            pltpu.SemaphoreType.DMA,
        ],
    )
    def gather_k(idx_hbm, w_hbm, out_hbm, idx_v, rows0, rows1,
                 sg0, sg1, ss0, ss1):
        wid = lax.axis_index("s") * _NC + lax.axis_index("c")
        out_base = wid * b_per_w

        pltpu.sync_copy(idx_hbm.at[pl.ds(out_base, b_per_w)], idx_v)

        rows = (rows0, rows1)
        sg = (sg0, sg1)
        ss = (ss0, ss1)

        def fire_gather(chunk, b):
            for j in range(NB):
                pltpu.async_copy(
                    w_hbm.at[idx_v.at[chunk * NB + j]],
                    rows[b].at[j],
                    sg[b],
                )

        def wait_gather(b):
            pltpu.make_async_copy(out_hbm.at[pl.ds(out_base, NB)],
                                  rows[b], sg[b]).wait()

        def fire_store(chunk, b):
            pltpu.async_copy(rows[b],
                             out_hbm.at[pl.ds(out_base + chunk * NB, NB)],
                             ss[b])

        def wait_store(b):
            pltpu.make_async_copy(rows[b], out_hbm.at[pl.ds(out_base, NB)],
                                  ss[b]).wait()

        fire_gather(0, 0)
        wait_gather(0)
        fire_store(0, 0)
        fire_gather(1, 1)

        def body(p, carry):
            i = 2 + 2 * p
            wait_store(0)
            fire_gather(i, 0)
            wait_gather(1)
            fire_store(i - 1, 1)
            wait_store(1)
            fire_gather(i + 1, 1)
            wait_gather(0)
            fire_store(i, 0)
            return carry

        lax.fori_loop(0, (n_chunks - 2) // 2, body, 0)

        wait_gather(1)
        fire_store(n_chunks - 1, 1)
        wait_store(0)
        wait_store(1)

    return gather_k


def kernel(input_, weight):
    BATCH, H = input_.shape
    return _make_gather(BATCH, H, weight.shape[0])(input_, weight)
